# Initial kernel scaffold; baseline (speedup 1.0000x reference)
import jax
import jax.numpy as jnp
from jax.experimental import pallas as pl


def kernel(x, edge_index, W_l1, b_l1, W_r1, W_l2, b_l2, W_r2):
    raise NotImplementedError("write your pallas kernel here")


# speedup vs baseline: 6.0743x; 6.0743x over previous
import functools

import jax
import jax.numpy as jnp
from jax import lax
from jax.experimental import pallas as pl
from jax.experimental.pallas import tpu as pltpu
from jax.experimental.pallas import tpu_sc as plsc

N = 10000
E = 320000
D_IN = 128
D_H = 64
D_OUT = 32

NC, NS = 2, 16
NW = NC * NS
MC = 128
KPW = 80
EPAD = NW * KPW * MC
ROWS_PT = 632
NPAD = NS * ROWS_PT

_MESH = dict(core_axis_name="c", subcore_axis_name="s", num_cores=NC,
             num_subcores=NS)


def _sc_agg_l1():
    scratch = [
        pltpu.VMEM((KPW, MC), jnp.int32),
        pltpu.VMEM((KPW, MC), jnp.int32),
        pltpu.VMEM((MC, D_H), jnp.float32),
        pltpu.VMEM((MC,), jnp.float32),
        pltpu.VMEM((ROWS_PT,), jnp.float32),
        pltpu.VMEM_SHARED((NPAD, D_H), jnp.float32),
        pltpu.VMEM_SHARED((NPAD,), jnp.float32),
        pltpu.SemaphoreType.DMA,
    ]
    out_type = (jax.ShapeDtypeStruct((NC, NPAD, D_H), jnp.float32),
                jax.ShapeDtypeStruct((NC * NPAD,), jnp.float32))

    @functools.partial(
        pl.kernel, out_type=out_type,
        mesh=plsc.VectorSubcoreMesh(**_MESH), scratch_types=scratch,
        compiler_params=pltpu.CompilerParams(use_tc_tiling_on_sc=False))
    def k(src_h, dst_h, y_h, z2_h, z1_h, agg_h, deg_h,
          srcv, dstv, rows, ones, dtmp, acc, dacc, sem):
        c = lax.axis_index("c")
        s = lax.axis_index("s")
        w = c * NS + s
        lo = s * ROWS_PT
        pltpu.sync_copy(z2_h.at[pl.ds(lo, ROWS_PT)], acc.at[pl.ds(lo, ROWS_PT)])
        pltpu.sync_copy(z1_h.at[pl.ds(lo, ROWS_PT)], dtmp)
        pltpu.sync_copy(dtmp, dacc.at[pl.ds(lo, ROWS_PT)])
        pltpu.sync_copy(src_h.at[pl.ds(w * KPW, KPW)], srcv)
        pltpu.sync_copy(dst_h.at[pl.ds(w * KPW, KPW)], dstv)
        for i in range(MC // 16):
            ones[pl.ds(i * 16, 16)] = jnp.ones((16,), jnp.float32)
        plsc.subcore_barrier()

        def step(j, carry):
            pltpu.async_copy(y_h.at[srcv.at[j]], rows, sem).wait()
            pltpu.sync_copy(rows, acc.at[dstv.at[j]], add=True)
            pltpu.sync_copy(ones, dacc.at[dstv.at[j]], add=True)
            return carry

        lax.fori_loop(0, KPW, step, 0)
        plsc.subcore_barrier()
        pltpu.sync_copy(acc.at[pl.ds(lo, ROWS_PT)],
                        agg_h.at[c, pl.ds(lo, ROWS_PT)])
        pltpu.sync_copy(dacc.at[pl.ds(lo, ROWS_PT)], dtmp)
        pltpu.sync_copy(dtmp, deg_h.at[pl.ds(c * NPAD + lo, ROWS_PT)])

    return k


def _sc_agg_l2():
    scratch = [
        pltpu.VMEM((KPW, MC), jnp.int32),
        pltpu.VMEM((KPW, MC), jnp.int32),
        pltpu.VMEM((MC, D_OUT), jnp.float32),
        pltpu.VMEM_SHARED((NPAD, D_OUT), jnp.float32),
        pltpu.SemaphoreType.DMA,
    ]
    out_type = jax.ShapeDtypeStruct((NC, NPAD, D_OUT), jnp.float32)

    @functools.partial(
        pl.kernel, out_type=out_type,
        mesh=plsc.VectorSubcoreMesh(**_MESH), scratch_types=scratch,
        compiler_params=pltpu.CompilerParams(use_tc_tiling_on_sc=False))
    def k(src_h, dst_h, y_h, z2_h, agg_h, srcv, dstv, rows, acc, sem):
        c = lax.axis_index("c")
        s = lax.axis_index("s")
        w = c * NS + s
        lo = s * ROWS_PT
        pltpu.sync_copy(z2_h.at[pl.ds(lo, ROWS_PT)], acc.at[pl.ds(lo, ROWS_PT)])
        pltpu.sync_copy(src_h.at[pl.ds(w * KPW, KPW)], srcv)
        pltpu.sync_copy(dst_h.at[pl.ds(w * KPW, KPW)], dstv)
        plsc.subcore_barrier()

        def step(j, carry):
            pltpu.async_copy(y_h.at[srcv.at[j]], rows, sem).wait()
            pltpu.sync_copy(rows, acc.at[dstv.at[j]], add=True)
            return carry

        lax.fori_loop(0, KPW, step, 0)
        plsc.subcore_barrier()
        pltpu.sync_copy(acc.at[pl.ds(lo, ROWS_PT)],
                        agg_h.at[c, pl.ds(lo, ROWS_PT)])

    return k


_DN = (((1,), (1,)), ((), ()))


def _mm1_body(x_ref, wl_ref, wr_ref, y_ref, r_ref):
    xb = x_ref[...]
    y_ref[...] = lax.dot_general(xb, wl_ref[...], _DN,
                                 preferred_element_type=jnp.float32)
    r_ref[...] = lax.dot_general(xb, wr_ref[...], _DN,
                                 preferred_element_type=jnp.float32)


def _comb1_body(a0, a1, d0, d1, xr, wl2, wr2, bl1, y2_ref, hr_ref):
    deg = d0[...] + d1[...]
    iv = 1.0 / jnp.maximum(deg, 1.0)
    h = (a0[...] + a1[...]) * iv + bl1[...] + xr[...]
    h = jnp.maximum(h, 0.0)
    y2_ref[...] = lax.dot_general(h, wl2[...], _DN,
                                  preferred_element_type=jnp.float32)
    hr_ref[...] = lax.dot_general(h, wr2[...], _DN,
                                  preferred_element_type=jnp.float32)


def _comb2_body(a0, a1, d0, d1, hr, bl2, out_ref):
    deg = d0[...] + d1[...]
    iv = 1.0 / jnp.maximum(deg, 1.0)
    out_ref[...] = (a0[...] + a1[...]) * iv + bl2[...] + hr[...]


_RB = 2000


def _row_spec(d):
    return pl.BlockSpec((_RB, d), lambda i: (i, 0))


def _full_spec(shape):
    nd = len(shape)
    return pl.BlockSpec(shape, lambda i, _n=nd: (0,) * _n)


def kernel(x, edge_index, W_l1, b_l1, W_r1, W_l2, b_l2, W_r2):
    pad = EPAD - E
    srcp = jnp.concatenate(
        [edge_index[0], jnp.zeros((pad,), jnp.int32)]).reshape(EPAD // MC, MC)
    dstp = jnp.concatenate(
        [edge_index[1], jnp.full((pad,), N, jnp.int32)]).reshape(EPAD // MC, MC)
    z2 = jnp.zeros((NPAD, D_H), jnp.float32)
    z1 = jnp.zeros((NPAD,), jnp.float32)
    z2b = jnp.zeros((NPAD, D_OUT), jnp.float32)

    y1, xr1 = pl.pallas_call(
        _mm1_body,
        grid=(N // _RB,),
        in_specs=[_row_spec(D_IN), _full_spec((D_H, D_IN)),
                  _full_spec((D_H, D_IN))],
        out_specs=[_row_spec(D_H), _row_spec(D_H)],
        out_shape=[jax.ShapeDtypeStruct((N, D_H), jnp.float32)] * 2,
    )(x, W_l1, W_r1)

    agg1, deg_flat = _sc_agg_l1()(srcp, dstp, y1, z2, z1)
    deg = deg_flat.reshape(NC, NPAD)
    d0 = deg[0, :N, None]
    d1 = deg[1, :N, None]

    y2, hr2 = pl.pallas_call(
        _comb1_body,
        grid=(N // _RB,),
        in_specs=[_row_spec(D_H), _row_spec(D_H),
                  pl.BlockSpec((_RB, 1), lambda i: (i, 0)),
                  pl.BlockSpec((_RB, 1), lambda i: (i, 0)),
                  _row_spec(D_H), _full_spec((D_OUT, D_H)),
                  _full_spec((D_OUT, D_H)), _full_spec((1, D_H))],
        out_specs=[_row_spec(D_OUT), _row_spec(D_OUT)],
        out_shape=[jax.ShapeDtypeStruct((N, D_OUT), jnp.float32)] * 2,
    )(agg1[0, :N], agg1[1, :N], d0, d1, xr1, W_l2, W_r2,
      b_l1.reshape(1, D_H))

    agg2 = _sc_agg_l2()(srcp, dstp, y2, z2b)

    out = pl.pallas_call(
        _comb2_body,
        grid=(N // _RB,),
        in_specs=[_row_spec(D_OUT), _row_spec(D_OUT),
                  pl.BlockSpec((_RB, 1), lambda i: (i, 0)),
                  pl.BlockSpec((_RB, 1), lambda i: (i, 0)),
                  _row_spec(D_OUT), _full_spec((1, D_OUT))],
        out_specs=_row_spec(D_OUT),
        out_shape=jax.ShapeDtypeStruct((N, D_OUT), jnp.float32),
    )(agg2[0, :N], agg2[1, :N], d0, d1, hr2, b_l2.reshape(1, D_OUT))

    return out


# speedup vs baseline: 7.2394x; 1.1918x over previous
import functools

import jax
import jax.numpy as jnp
from jax import lax
from jax.experimental import pallas as pl
from jax.experimental.pallas import tpu as pltpu
from jax.experimental.pallas import tpu_sc as plsc

N = 10000
E = 320000
D_IN = 128
D_H = 64
D_OUT = 32

NC, NS = 2, 16
NW = NC * NS
MC = 128
KPW = 80
EPAD = NW * KPW * MC
ROWS_PT = 632
NPAD = NS * ROWS_PT

_MESH = dict(core_axis_name="c", subcore_axis_name="s", num_cores=NC,
             num_subcores=NS)
NBUF = 4


def _sc_agg_l1():
    scratch = [
        pltpu.VMEM((KPW, MC), jnp.int32),
        pltpu.VMEM((KPW, MC), jnp.int32),
        pltpu.VMEM((NBUF, MC, D_H), jnp.float32),
        pltpu.VMEM((MC,), jnp.float32),
        pltpu.VMEM((ROWS_PT,), jnp.float32),
        pltpu.VMEM_SHARED((NPAD, D_H), jnp.float32),
        pltpu.VMEM_SHARED((NPAD,), jnp.float32),
        [pltpu.SemaphoreType.DMA] * NBUF,
    ]
    out_type = (jax.ShapeDtypeStruct((NC, NPAD, D_H), jnp.float32),
                jax.ShapeDtypeStruct((NC * NPAD,), jnp.float32))

    @functools.partial(
        pl.kernel, out_type=out_type,
        mesh=plsc.VectorSubcoreMesh(**_MESH), scratch_types=scratch,
        compiler_params=pltpu.CompilerParams(use_tc_tiling_on_sc=False))
    def k(src_h, dst_h, y_h, z2_h, z1_h, agg_h, deg_h,
          srcv, dstv, rows, ones, dtmp, acc, dacc, sems):
        c = lax.axis_index("c")
        s = lax.axis_index("s")
        w = c * NS + s
        lo = s * ROWS_PT
        pltpu.sync_copy(z2_h.at[pl.ds(lo, ROWS_PT)], acc.at[pl.ds(lo, ROWS_PT)])
        pltpu.sync_copy(z1_h.at[pl.ds(lo, ROWS_PT)], dtmp)
        pltpu.sync_copy(dtmp, dacc.at[pl.ds(lo, ROWS_PT)])
        pltpu.sync_copy(src_h.at[pl.ds(w * KPW, KPW)], srcv)
        pltpu.sync_copy(dst_h.at[pl.ds(w * KPW, KPW)], dstv)
        for i in range(MC // 16):
            ones[pl.ds(i * 16, 16)] = jnp.ones((16,), jnp.float32)
        plsc.subcore_barrier()

        for b in range(NBUF):
            pltpu.async_copy(y_h.at[srcv.at[b]], rows.at[b], sems[b])

        def group(g, carry):
            for b in range(NBUF):
                j = g * NBUF + b
                pltpu.make_async_copy(y_h.at[pl.ds(0, MC)], rows.at[b],
                                      sems[b]).wait()
                pltpu.sync_copy(rows.at[b], acc.at[dstv.at[j]], add=True)
                pltpu.sync_copy(ones, dacc.at[dstv.at[j]], add=True)

                @pl.when(j + NBUF < KPW)
                def _():
                    pltpu.async_copy(y_h.at[srcv.at[j + NBUF]], rows.at[b],
                                     sems[b])
            return carry

        lax.fori_loop(0, KPW // NBUF, group, 0)
        plsc.subcore_barrier()
        pltpu.sync_copy(acc.at[pl.ds(lo, ROWS_PT)],
                        agg_h.at[c, pl.ds(lo, ROWS_PT)])
        pltpu.sync_copy(dacc.at[pl.ds(lo, ROWS_PT)], dtmp)
        pltpu.sync_copy(dtmp, deg_h.at[pl.ds(c * NPAD + lo, ROWS_PT)])

    return k


def _sc_agg_l2():
    scratch = [
        pltpu.VMEM((KPW, MC), jnp.int32),
        pltpu.VMEM((KPW, MC), jnp.int32),
        pltpu.VMEM((NBUF, MC, D_OUT), jnp.float32),
        pltpu.VMEM_SHARED((NPAD, D_OUT), jnp.float32),
        [pltpu.SemaphoreType.DMA] * NBUF,
    ]
    out_type = jax.ShapeDtypeStruct((NC, NPAD, D_OUT), jnp.float32)

    @functools.partial(
        pl.kernel, out_type=out_type,
        mesh=plsc.VectorSubcoreMesh(**_MESH), scratch_types=scratch,
        compiler_params=pltpu.CompilerParams(use_tc_tiling_on_sc=False))
    def k(src_h, dst_h, y_h, z2_h, agg_h, srcv, dstv, rows, acc, sems):
        c = lax.axis_index("c")
        s = lax.axis_index("s")
        w = c * NS + s
        lo = s * ROWS_PT
        pltpu.sync_copy(z2_h.at[pl.ds(lo, ROWS_PT)], acc.at[pl.ds(lo, ROWS_PT)])
        pltpu.sync_copy(src_h.at[pl.ds(w * KPW, KPW)], srcv)
        pltpu.sync_copy(dst_h.at[pl.ds(w * KPW, KPW)], dstv)
        plsc.subcore_barrier()

        for b in range(NBUF):
            pltpu.async_copy(y_h.at[srcv.at[b]], rows.at[b], sems[b])

        def group(g, carry):
            for b in range(NBUF):
                j = g * NBUF + b
                pltpu.make_async_copy(y_h.at[pl.ds(0, MC)], rows.at[b],
                                      sems[b]).wait()
                pltpu.sync_copy(rows.at[b], acc.at[dstv.at[j]], add=True)

                @pl.when(j + NBUF < KPW)
                def _():
                    pltpu.async_copy(y_h.at[srcv.at[j + NBUF]], rows.at[b],
                                     sems[b])
            return carry

        lax.fori_loop(0, KPW // NBUF, group, 0)
        plsc.subcore_barrier()
        pltpu.sync_copy(acc.at[pl.ds(lo, ROWS_PT)],
                        agg_h.at[c, pl.ds(lo, ROWS_PT)])

    return k


_DN = (((1,), (1,)), ((), ()))


def _mm1_body(x_ref, wl_ref, wr_ref, y_ref, r_ref):
    xb = x_ref[...]
    y_ref[...] = lax.dot_general(xb, wl_ref[...], _DN,
                                 preferred_element_type=jnp.float32)
    r_ref[...] = lax.dot_general(xb, wr_ref[...], _DN,
                                 preferred_element_type=jnp.float32)


def _comb1_body(a0, a1, d0, d1, xr, wl2, wr2, bl1, y2_ref, hr_ref):
    deg = d0[...] + d1[...]
    iv = 1.0 / jnp.maximum(deg, 1.0)
    h = (a0[...] + a1[...]) * iv + bl1[...] + xr[...]
    h = jnp.maximum(h, 0.0)
    y2_ref[...] = lax.dot_general(h, wl2[...], _DN,
                                  preferred_element_type=jnp.float32)
    hr_ref[...] = lax.dot_general(h, wr2[...], _DN,
                                  preferred_element_type=jnp.float32)


def _comb2_body(a0, a1, d0, d1, hr, bl2, out_ref):
    deg = d0[...] + d1[...]
    iv = 1.0 / jnp.maximum(deg, 1.0)
    out_ref[...] = (a0[...] + a1[...]) * iv + bl2[...] + hr[...]


_RB = 2000


def _row_spec(d):
    return pl.BlockSpec((_RB, d), lambda i: (i, 0))


def _full_spec(shape):
    nd = len(shape)
    return pl.BlockSpec(shape, lambda i, _n=nd: (0,) * _n)


def kernel(x, edge_index, W_l1, b_l1, W_r1, W_l2, b_l2, W_r2):
    pad = EPAD - E
    srcp = jnp.concatenate(
        [edge_index[0], jnp.zeros((pad,), jnp.int32)]).reshape(EPAD // MC, MC)
    dstp = jnp.concatenate(
        [edge_index[1], jnp.full((pad,), N, jnp.int32)]).reshape(EPAD // MC, MC)
    z2 = jnp.zeros((NPAD, D_H), jnp.float32)
    z1 = jnp.zeros((NPAD,), jnp.float32)
    z2b = jnp.zeros((NPAD, D_OUT), jnp.float32)

    y1, xr1 = pl.pallas_call(
        _mm1_body,
        grid=(N // _RB,),
        in_specs=[_row_spec(D_IN), _full_spec((D_H, D_IN)),
                  _full_spec((D_H, D_IN))],
        out_specs=[_row_spec(D_H), _row_spec(D_H)],
        out_shape=[jax.ShapeDtypeStruct((N, D_H), jnp.float32)] * 2,
    )(x, W_l1, W_r1)

    agg1, deg_flat = _sc_agg_l1()(srcp, dstp, y1, z2, z1)
    deg = deg_flat.reshape(NC, NPAD)
    d0 = deg[0, :N, None]
    d1 = deg[1, :N, None]

    y2, hr2 = pl.pallas_call(
        _comb1_body,
        grid=(N // _RB,),
        in_specs=[_row_spec(D_H), _row_spec(D_H),
                  pl.BlockSpec((_RB, 1), lambda i: (i, 0)),
                  pl.BlockSpec((_RB, 1), lambda i: (i, 0)),
                  _row_spec(D_H), _full_spec((D_OUT, D_H)),
                  _full_spec((D_OUT, D_H)), _full_spec((1, D_H))],
        out_specs=[_row_spec(D_OUT), _row_spec(D_OUT)],
        out_shape=[jax.ShapeDtypeStruct((N, D_OUT), jnp.float32)] * 2,
    )(agg1[0, :N], agg1[1, :N], d0, d1, xr1, W_l2, W_r2,
      b_l1.reshape(1, D_H))

    agg2 = _sc_agg_l2()(srcp, dstp, y2, z2b)

    out = pl.pallas_call(
        _comb2_body,
        grid=(N // _RB,),
        in_specs=[_row_spec(D_OUT), _row_spec(D_OUT),
                  pl.BlockSpec((_RB, 1), lambda i: (i, 0)),
                  pl.BlockSpec((_RB, 1), lambda i: (i, 0)),
                  _row_spec(D_OUT), _full_spec((1, D_OUT))],
        out_specs=_row_spec(D_OUT),
        out_shape=jax.ShapeDtypeStruct((N, D_OUT), jnp.float32),
    )(agg2[0, :N], agg2[1, :N], d0, d1, hr2, b_l2.reshape(1, D_OUT))

    return out


# speedup vs baseline: 8.3462x; 1.1529x over previous
import functools

import jax
import jax.numpy as jnp
from jax import lax
from jax.experimental import pallas as pl
from jax.experimental.pallas import tpu as pltpu
from jax.experimental.pallas import tpu_sc as plsc

N = 10000
E = 320000
D_IN = 128
D_H = 64
D_OUT = 32

NC, NS = 2, 16
NW = NC * NS
MC = 128
KPW = 80
EPAD = NW * KPW * MC
ROWS_PT = 632
NPAD = NS * ROWS_PT

_MESH = dict(core_axis_name="c", subcore_axis_name="s", num_cores=NC,
             num_subcores=NS)
NBUF = 4


def _sc_agg_l1():
    scratch = [
        pltpu.VMEM((KPW, MC), jnp.int32),
        pltpu.VMEM((KPW, MC), jnp.int32),
        pltpu.VMEM((NBUF, MC, D_H), jnp.float32),
        pltpu.VMEM((MC,), jnp.float32),
        pltpu.VMEM((ROWS_PT,), jnp.float32),
        pltpu.VMEM_SHARED((NPAD, D_H), jnp.float32),
        pltpu.VMEM_SHARED((NPAD,), jnp.float32),
        [pltpu.SemaphoreType.DMA] * NBUF,
    ]
    out_type = (jax.ShapeDtypeStruct((NC, NPAD, D_H), jnp.float32),
                jax.ShapeDtypeStruct((NC * NPAD,), jnp.float32))

    @functools.partial(
        pl.kernel, out_type=out_type,
        mesh=plsc.VectorSubcoreMesh(**_MESH), scratch_types=scratch,
        compiler_params=pltpu.CompilerParams(use_tc_tiling_on_sc=False))
    def k(src_h, dst_h, y_h, z2_h, z1_h, agg_h, deg_h,
          srcv, dstv, rows, ones, dtmp, acc, dacc, sems):
        c = lax.axis_index("c")
        s = lax.axis_index("s")
        w = c * NS + s
        lo = s * ROWS_PT
        pltpu.sync_copy(z2_h.at[pl.ds(lo, ROWS_PT)], acc.at[pl.ds(lo, ROWS_PT)])
        pltpu.sync_copy(z1_h.at[pl.ds(lo, ROWS_PT)], dtmp)
        pltpu.sync_copy(dtmp, dacc.at[pl.ds(lo, ROWS_PT)])
        pltpu.sync_copy(src_h.at[pl.ds(w * KPW, KPW)], srcv)
        pltpu.sync_copy(dst_h.at[pl.ds(w * KPW, KPW)], dstv)
        for i in range(MC // 16):
            ones[pl.ds(i * 16, 16)] = jnp.ones((16,), jnp.float32)
        plsc.subcore_barrier()

        for b in range(NBUF):
            pltpu.async_copy(y_h.at[srcv.at[b]], rows.at[b], sems[b])

        def group(g, carry):
            for b in range(NBUF):
                j = g * NBUF + b
                pltpu.make_async_copy(y_h.at[pl.ds(0, MC)], rows.at[b],
                                      sems[b]).wait()
                pltpu.sync_copy(rows.at[b], acc.at[dstv.at[j]], add=True)
                pltpu.sync_copy(ones, dacc.at[dstv.at[j]], add=True)

                @pl.when(j + NBUF < KPW)
                def _():
                    pltpu.async_copy(y_h.at[srcv.at[j + NBUF]], rows.at[b],
                                     sems[b])
            return carry

        lax.fori_loop(0, KPW // NBUF, group, 0)
        plsc.subcore_barrier()
        pltpu.sync_copy(acc.at[pl.ds(lo, ROWS_PT)],
                        agg_h.at[c, pl.ds(lo, ROWS_PT)])
        pltpu.sync_copy(dacc.at[pl.ds(lo, ROWS_PT)], dtmp)
        pltpu.sync_copy(dtmp, deg_h.at[pl.ds(c * NPAD + lo, ROWS_PT)])

    return k


def _sc_agg_l2():
    scratch = [
        pltpu.VMEM((KPW, MC), jnp.int32),
        pltpu.VMEM((KPW, MC), jnp.int32),
        pltpu.VMEM((NBUF, MC, D_OUT), jnp.float32),
        pltpu.VMEM_SHARED((NPAD, D_OUT), jnp.float32),
        pltpu.VMEM_SHARED((N, D_OUT), jnp.float32),
        [pltpu.SemaphoreType.DMA] * NBUF,
    ]
    out_type = jax.ShapeDtypeStruct((NC, NPAD, D_OUT), jnp.float32)

    @functools.partial(
        pl.kernel, out_type=out_type,
        mesh=plsc.VectorSubcoreMesh(**_MESH), scratch_types=scratch,
        compiler_params=pltpu.CompilerParams(use_tc_tiling_on_sc=False))
    def k(src_h, dst_h, y_h, z2_h, agg_h, srcv, dstv, rows, acc, y_s, sems):
        c = lax.axis_index("c")
        s = lax.axis_index("s")
        w = c * NS + s
        lo = s * ROWS_PT
        yl = s * (N // NS)
        pltpu.sync_copy(z2_h.at[pl.ds(lo, ROWS_PT)], acc.at[pl.ds(lo, ROWS_PT)])
        pltpu.sync_copy(y_h.at[pl.ds(yl, N // NS)], y_s.at[pl.ds(yl, N // NS)])
        pltpu.sync_copy(src_h.at[pl.ds(w * KPW, KPW)], srcv)
        pltpu.sync_copy(dst_h.at[pl.ds(w * KPW, KPW)], dstv)
        plsc.subcore_barrier()

        for b in range(NBUF):
            pltpu.async_copy(y_s.at[srcv.at[b]], rows.at[b], sems[b])

        def group(g, carry):
            for b in range(NBUF):
                j = g * NBUF + b
                pltpu.make_async_copy(y_s.at[pl.ds(0, MC)], rows.at[b],
                                      sems[b]).wait()
                pltpu.sync_copy(rows.at[b], acc.at[dstv.at[j]], add=True)

                @pl.when(j + NBUF < KPW)
                def _():
                    pltpu.async_copy(y_s.at[srcv.at[j + NBUF]], rows.at[b],
                                     sems[b])
            return carry

        lax.fori_loop(0, KPW // NBUF, group, 0)
        plsc.subcore_barrier()
        pltpu.sync_copy(acc.at[pl.ds(lo, ROWS_PT)],
                        agg_h.at[c, pl.ds(lo, ROWS_PT)])

    return k


_DN = (((1,), (1,)), ((), ()))


def _mm1_body(x_ref, wl_ref, wr_ref, y_ref, r_ref):
    xb = x_ref[...]
    y_ref[...] = lax.dot_general(xb, wl_ref[...], _DN,
                                 preferred_element_type=jnp.float32)
    r_ref[...] = lax.dot_general(xb, wr_ref[...], _DN,
                                 preferred_element_type=jnp.float32)


def _comb1_body(a0, a1, d0, d1, xr, wl2, wr2, bl1, y2_ref, hr_ref):
    deg = d0[...] + d1[...]
    iv = 1.0 / jnp.maximum(deg, 1.0)
    h = (a0[...] + a1[...]) * iv + bl1[...] + xr[...]
    h = jnp.maximum(h, 0.0)
    y2_ref[...] = lax.dot_general(h, wl2[...], _DN,
                                  preferred_element_type=jnp.float32)
    hr_ref[...] = lax.dot_general(h, wr2[...], _DN,
                                  preferred_element_type=jnp.float32)


def _comb2_body(a0, a1, d0, d1, hr, bl2, out_ref):
    deg = d0[...] + d1[...]
    iv = 1.0 / jnp.maximum(deg, 1.0)
    out_ref[...] = (a0[...] + a1[...]) * iv + bl2[...] + hr[...]


_RB = 2000


def _row_spec(d):
    return pl.BlockSpec((_RB, d), lambda i: (i, 0))


def _full_spec(shape):
    nd = len(shape)
    return pl.BlockSpec(shape, lambda i, _n=nd: (0,) * _n)


def kernel(x, edge_index, W_l1, b_l1, W_r1, W_l2, b_l2, W_r2):
    pad = EPAD - E
    srcp = jnp.concatenate(
        [edge_index[0], jnp.zeros((pad,), jnp.int32)]).reshape(EPAD // MC, MC)
    dstp = jnp.concatenate(
        [edge_index[1], jnp.full((pad,), N, jnp.int32)]).reshape(EPAD // MC, MC)
    z2 = jnp.zeros((NPAD, D_H), jnp.float32)
    z1 = jnp.zeros((NPAD,), jnp.float32)
    z2b = jnp.zeros((NPAD, D_OUT), jnp.float32)

    y1, xr1 = pl.pallas_call(
        _mm1_body,
        grid=(N // _RB,),
        in_specs=[_row_spec(D_IN), _full_spec((D_H, D_IN)),
                  _full_spec((D_H, D_IN))],
        out_specs=[_row_spec(D_H), _row_spec(D_H)],
        out_shape=[jax.ShapeDtypeStruct((N, D_H), jnp.float32)] * 2,
    )(x, W_l1, W_r1)

    agg1, deg_flat = _sc_agg_l1()(srcp, dstp, y1, z2, z1)
    deg = deg_flat.reshape(NC, NPAD)
    d0 = deg[0, :N, None]
    d1 = deg[1, :N, None]

    y2, hr2 = pl.pallas_call(
        _comb1_body,
        grid=(N // _RB,),
        in_specs=[_row_spec(D_H), _row_spec(D_H),
                  pl.BlockSpec((_RB, 1), lambda i: (i, 0)),
                  pl.BlockSpec((_RB, 1), lambda i: (i, 0)),
                  _row_spec(D_H), _full_spec((D_OUT, D_H)),
                  _full_spec((D_OUT, D_H)), _full_spec((1, D_H))],
        out_specs=[_row_spec(D_OUT), _row_spec(D_OUT)],
        out_shape=[jax.ShapeDtypeStruct((N, D_OUT), jnp.float32)] * 2,
    )(agg1[0, :N], agg1[1, :N], d0, d1, xr1, W_l2, W_r2,
      b_l1.reshape(1, D_H))

    agg2 = _sc_agg_l2()(srcp, dstp, y2, z2b)

    out = pl.pallas_call(
        _comb2_body,
        grid=(N // _RB,),
        in_specs=[_row_spec(D_OUT), _row_spec(D_OUT),
                  pl.BlockSpec((_RB, 1), lambda i: (i, 0)),
                  pl.BlockSpec((_RB, 1), lambda i: (i, 0)),
                  _row_spec(D_OUT), _full_spec((1, D_OUT))],
        out_specs=_row_spec(D_OUT),
        out_shape=jax.ShapeDtypeStruct((N, D_OUT), jnp.float32),
    )(agg2[0, :N], agg2[1, :N], d0, d1, hr2, b_l2.reshape(1, D_OUT))

    return out


# speedup vs baseline: 8.7627x; 1.0499x over previous
import functools

import jax
import jax.numpy as jnp
from jax import lax
from jax.experimental import pallas as pl
from jax.experimental.pallas import tpu as pltpu
from jax.experimental.pallas import tpu_sc as plsc

N = 10000
E = 320000
D_IN = 128
D_H = 64
D_OUT = 32

NC, NS = 2, 16
NW = NC * NS
MC = 128
KPW = 80
NCHUNKS = NW * KPW
KPW0 = 132
KPW1 = 28
NCHUNKS_PAD = NS * KPW0 + (NS - 1) * KPW1 + KPW0
EPAD = NCHUNKS_PAD * MC
ROWS_PT = 632
NPAD = NS * ROWS_PT

_MESH = dict(core_axis_name="c", subcore_axis_name="s", num_cores=NC,
             num_subcores=NS)
NBUF = 4


def _sc_agg_l1():
    scratch = [
        pltpu.VMEM((KPW0, MC), jnp.int32),
        pltpu.VMEM((KPW0, MC), jnp.int32),
        pltpu.VMEM((NBUF, MC, D_H), jnp.float32),
        pltpu.VMEM((MC,), jnp.float32),
        pltpu.VMEM((ROWS_PT,), jnp.float32),
        pltpu.VMEM_SHARED((NPAD, D_H), jnp.float32),
        pltpu.VMEM_SHARED((NPAD,), jnp.float32),
        [pltpu.SemaphoreType.DMA] * NBUF,
    ]
    out_type = (jax.ShapeDtypeStruct((NC, NPAD, D_H), jnp.float32),
                jax.ShapeDtypeStruct((NC * NPAD,), jnp.float32))

    @functools.partial(
        pl.kernel, out_type=out_type,
        mesh=plsc.VectorSubcoreMesh(**_MESH), scratch_types=scratch,
        compiler_params=pltpu.CompilerParams(use_tc_tiling_on_sc=False))
    def k(src_h, dst_h, y_h, z2_h, z1_h, agg_h, deg_h,
          srcv, dstv, rows, ones, dtmp, acc, dacc, sems):
        c = lax.axis_index("c")
        s = lax.axis_index("s")
        lo = s * ROWS_PT
        base = jnp.where(c == 0, s * KPW0, NS * KPW0 + s * KPW1)
        nck = jnp.where(c == 0, KPW0, KPW1)
        pltpu.sync_copy(z2_h.at[pl.ds(lo, ROWS_PT)], acc.at[pl.ds(lo, ROWS_PT)])
        pltpu.sync_copy(z1_h.at[pl.ds(lo, ROWS_PT)], dtmp)
        pltpu.sync_copy(dtmp, dacc.at[pl.ds(lo, ROWS_PT)])
        pltpu.sync_copy(src_h.at[pl.ds(base, KPW0)], srcv)
        pltpu.sync_copy(dst_h.at[pl.ds(base, KPW0)], dstv)
        for i in range(MC // 16):
            ones[pl.ds(i * 16, 16)] = jnp.ones((16,), jnp.float32)
        plsc.subcore_barrier()

        for b in range(NBUF):
            pltpu.async_copy(y_h.at[srcv.at[b]], rows.at[b], sems[b])

        def group(g, carry):
            for b in range(NBUF):
                j = g * NBUF + b
                pltpu.make_async_copy(y_h.at[pl.ds(0, MC)], rows.at[b],
                                      sems[b]).wait()
                pltpu.sync_copy(rows.at[b], acc.at[dstv.at[j]], add=True)
                pltpu.sync_copy(ones, dacc.at[dstv.at[j]], add=True)

                @pl.when(j + NBUF < nck)
                def _():
                    pltpu.async_copy(y_h.at[srcv.at[j + NBUF]], rows.at[b],
                                     sems[b])
            return carry

        lax.fori_loop(0, nck // NBUF, group, 0)
        plsc.subcore_barrier()
        pltpu.sync_copy(acc.at[pl.ds(lo, ROWS_PT)],
                        agg_h.at[c, pl.ds(lo, ROWS_PT)])
        pltpu.sync_copy(dacc.at[pl.ds(lo, ROWS_PT)], dtmp)
        pltpu.sync_copy(dtmp, deg_h.at[pl.ds(c * NPAD + lo, ROWS_PT)])

    return k


def _sc_agg_l2():
    scratch = [
        pltpu.VMEM((KPW, MC), jnp.int32),
        pltpu.VMEM((KPW, MC), jnp.int32),
        pltpu.VMEM((NBUF, MC, D_OUT), jnp.float32),
        pltpu.VMEM_SHARED((NPAD, D_OUT), jnp.float32),
        pltpu.VMEM_SHARED((N, D_OUT), jnp.float32),
        [pltpu.SemaphoreType.DMA] * NBUF,
    ]
    out_type = jax.ShapeDtypeStruct((NC, NPAD, D_OUT), jnp.float32)

    @functools.partial(
        pl.kernel, out_type=out_type,
        mesh=plsc.VectorSubcoreMesh(**_MESH), scratch_types=scratch,
        compiler_params=pltpu.CompilerParams(use_tc_tiling_on_sc=False))
    def k(src_h, dst_h, y_h, z2_h, agg_h, srcv, dstv, rows, acc, y_s, sems):
        c = lax.axis_index("c")
        s = lax.axis_index("s")
        w = c * NS + s
        lo = s * ROWS_PT
        yl = s * (N // NS)
        pltpu.sync_copy(z2_h.at[pl.ds(lo, ROWS_PT)], acc.at[pl.ds(lo, ROWS_PT)])
        pltpu.sync_copy(y_h.at[pl.ds(yl, N // NS)], y_s.at[pl.ds(yl, N // NS)])
        pltpu.sync_copy(src_h.at[pl.ds(w * KPW, KPW)], srcv)
        pltpu.sync_copy(dst_h.at[pl.ds(w * KPW, KPW)], dstv)
        plsc.subcore_barrier()

        for b in range(NBUF):
            pltpu.async_copy(y_s.at[srcv.at[b]], rows.at[b], sems[b])

        def group(g, carry):
            for b in range(NBUF):
                j = g * NBUF + b
                pltpu.make_async_copy(y_s.at[pl.ds(0, MC)], rows.at[b],
                                      sems[b]).wait()
                pltpu.sync_copy(rows.at[b], acc.at[dstv.at[j]], add=True)

                @pl.when(j + NBUF < KPW)
                def _():
                    pltpu.async_copy(y_s.at[srcv.at[j + NBUF]], rows.at[b],
                                     sems[b])
            return carry

        lax.fori_loop(0, KPW // NBUF, group, 0)
        plsc.subcore_barrier()
        pltpu.sync_copy(acc.at[pl.ds(lo, ROWS_PT)],
                        agg_h.at[c, pl.ds(lo, ROWS_PT)])

    return k


_DN = (((1,), (1,)), ((), ()))


def _mm1_body(x_ref, wl_ref, wr_ref, y_ref, r_ref):
    xb = x_ref[...]
    y_ref[...] = lax.dot_general(xb, wl_ref[...], _DN,
                                 preferred_element_type=jnp.float32)
    r_ref[...] = lax.dot_general(xb, wr_ref[...], _DN,
                                 preferred_element_type=jnp.float32)


def _comb1_body(a0, a1, d0, d1, xr, wl2, wr2, bl1, y2_ref, hr_ref):
    deg = d0[...] + d1[...]
    iv = 1.0 / jnp.maximum(deg, 1.0)
    h = (a0[...] + a1[...]) * iv + bl1[...] + xr[...]
    h = jnp.maximum(h, 0.0)
    y2_ref[...] = lax.dot_general(h, wl2[...], _DN,
                                  preferred_element_type=jnp.float32)
    hr_ref[...] = lax.dot_general(h, wr2[...], _DN,
                                  preferred_element_type=jnp.float32)


def _comb2_body(a0, a1, d0, d1, hr, bl2, out_ref):
    deg = d0[...] + d1[...]
    iv = 1.0 / jnp.maximum(deg, 1.0)
    out_ref[...] = (a0[...] + a1[...]) * iv + bl2[...] + hr[...]


_RB = 2000


def _row_spec(d):
    return pl.BlockSpec((_RB, d), lambda i: (i, 0))


def _full_spec(shape):
    nd = len(shape)
    return pl.BlockSpec(shape, lambda i, _n=nd: (0,) * _n)


def kernel(x, edge_index, W_l1, b_l1, W_r1, W_l2, b_l2, W_r2):
    pad = EPAD - E
    srcp = jnp.concatenate(
        [edge_index[0], jnp.zeros((pad,), jnp.int32)]).reshape(NCHUNKS_PAD, MC)
    dstp = jnp.concatenate(
        [edge_index[1], jnp.full((pad,), N, jnp.int32)]).reshape(NCHUNKS_PAD, MC)
    z2 = jnp.zeros((NPAD, D_H), jnp.float32)
    z1 = jnp.zeros((NPAD,), jnp.float32)
    z2b = jnp.zeros((NPAD, D_OUT), jnp.float32)

    y1, xr1 = pl.pallas_call(
        _mm1_body,
        grid=(N // _RB,),
        in_specs=[_row_spec(D_IN), _full_spec((D_H, D_IN)),
                  _full_spec((D_H, D_IN))],
        out_specs=[_row_spec(D_H), _row_spec(D_H)],
        out_shape=[jax.ShapeDtypeStruct((N, D_H), jnp.float32)] * 2,
    )(x, W_l1, W_r1)

    agg1, deg_flat = _sc_agg_l1()(srcp, dstp, y1, z2, z1)
    deg = deg_flat.reshape(NC, NPAD)
    d0 = deg[0, :N, None]
    d1 = deg[1, :N, None]

    y2, hr2 = pl.pallas_call(
        _comb1_body,
        grid=(N // _RB,),
        in_specs=[_row_spec(D_H), _row_spec(D_H),
                  pl.BlockSpec((_RB, 1), lambda i: (i, 0)),
                  pl.BlockSpec((_RB, 1), lambda i: (i, 0)),
                  _row_spec(D_H), _full_spec((D_OUT, D_H)),
                  _full_spec((D_OUT, D_H)), _full_spec((1, D_H))],
        out_specs=[_row_spec(D_OUT), _row_spec(D_OUT)],
        out_shape=[jax.ShapeDtypeStruct((N, D_OUT), jnp.float32)] * 2,
    )(agg1[0, :N], agg1[1, :N], d0, d1, xr1, W_l2, W_r2,
      b_l1.reshape(1, D_H))

    agg2 = _sc_agg_l2()(srcp, dstp, y2, z2b)

    out = pl.pallas_call(
        _comb2_body,
        grid=(N // _RB,),
        in_specs=[_row_spec(D_OUT), _row_spec(D_OUT),
                  pl.BlockSpec((_RB, 1), lambda i: (i, 0)),
                  pl.BlockSpec((_RB, 1), lambda i: (i, 0)),
                  _row_spec(D_OUT), _full_spec((1, D_OUT))],
        out_specs=_row_spec(D_OUT),
        out_shape=jax.ShapeDtypeStruct((N, D_OUT), jnp.float32),
    )(agg2[0, :N], agg2[1, :N], d0, d1, hr2, b_l2.reshape(1, D_OUT))

    return out


# speedup vs baseline: 8.9598x; 1.0225x over previous
import functools

import jax
import jax.numpy as jnp
from jax import lax
from jax.experimental import pallas as pl
from jax.experimental.pallas import tpu as pltpu
from jax.experimental.pallas import tpu_sc as plsc

N = 10000
E = 320000
D_IN = 128
D_H = 64
D_OUT = 32

NC, NS = 2, 16
NW = NC * NS
MC = 128
KPW = 80
NCHUNKS = NW * KPW
KPW0 = 132
KPW1 = 28
NCHUNKS_PAD = NS * KPW0 + (NS - 1) * KPW1 + KPW0
EPAD = NCHUNKS_PAD * MC
ROWS_PT = 632
NPAD = NS * ROWS_PT
NBUF = 4

_MESH = dict(core_axis_name="c", subcore_axis_name="s", num_cores=NC,
             num_subcores=NS)


def _sc_agg_l1():
    scratch = [
        pltpu.VMEM((KPW0, MC), jnp.int32),
        pltpu.VMEM((KPW0, MC), jnp.int32),
        pltpu.VMEM((NBUF, MC, D_H), jnp.float32),
        pltpu.VMEM((MC,), jnp.float32),
        pltpu.VMEM((ROWS_PT,), jnp.float32),
        pltpu.VMEM_SHARED((NPAD, D_H), jnp.float32),
        pltpu.VMEM_SHARED((NPAD,), jnp.float32),
        [pltpu.SemaphoreType.DMA] * NBUF,
    ]
    out_type = (jax.ShapeDtypeStruct((NC, NPAD, D_H), jnp.float32),
                jax.ShapeDtypeStruct((NC * NPAD,), jnp.float32))

    @functools.partial(
        pl.kernel, out_type=out_type,
        mesh=plsc.VectorSubcoreMesh(**_MESH), scratch_types=scratch,
        compiler_params=pltpu.CompilerParams(use_tc_tiling_on_sc=False))
    def k(src_h, dst_h, y_h, z2_h, z1_h, agg_h, deg_h,
          srcv, dstv, rows, ones, dtmp, acc, dacc, sems):
        c = lax.axis_index("c")
        s = lax.axis_index("s")
        lo = s * ROWS_PT
        base = jnp.where(c == 0, s * KPW0, NS * KPW0 + s * KPW1)
        nck = jnp.where(c == 0, KPW0, KPW1)
        pltpu.sync_copy(z2_h.at[pl.ds(lo, ROWS_PT)], acc.at[pl.ds(lo, ROWS_PT)])
        pltpu.sync_copy(z1_h.at[pl.ds(lo, ROWS_PT)], dtmp)
        pltpu.sync_copy(dtmp, dacc.at[pl.ds(lo, ROWS_PT)])
        pltpu.sync_copy(src_h.at[pl.ds(base, KPW0)], srcv)
        pltpu.sync_copy(dst_h.at[pl.ds(base, KPW0)], dstv)
        for i in range(MC // 16):
            ones[pl.ds(i * 16, 16)] = jnp.ones((16,), jnp.float32)
        plsc.subcore_barrier()

        for b in range(NBUF):
            pltpu.async_copy(y_h.at[srcv.at[b]], rows.at[b], sems[b])

        def group(g, carry):
            for b in range(NBUF):
                j = g * NBUF + b
                pltpu.make_async_copy(y_h.at[pl.ds(0, MC)], rows.at[b],
                                      sems[b]).wait()
                pltpu.sync_copy(rows.at[b], acc.at[dstv.at[j]], add=True)
                pltpu.sync_copy(ones, dacc.at[dstv.at[j]], add=True)

                @pl.when(j + NBUF < nck)
                def _():
                    pltpu.async_copy(y_h.at[srcv.at[j + NBUF]], rows.at[b],
                                     sems[b])
            return carry

        lax.fori_loop(0, nck // NBUF, group, 0)
        plsc.subcore_barrier()
        pltpu.sync_copy(acc.at[pl.ds(lo, ROWS_PT)],
                        agg_h.at[c, pl.ds(lo, ROWS_PT)])
        pltpu.sync_copy(dacc.at[pl.ds(lo, ROWS_PT)], dtmp)
        pltpu.sync_copy(dtmp, deg_h.at[pl.ds(c * NPAD + lo, ROWS_PT)])

    return k


def _sc_agg_l2():
    scratch = [
        pltpu.VMEM((KPW, MC), jnp.int32),
        pltpu.VMEM((KPW, MC), jnp.int32),
        pltpu.VMEM((NBUF, MC, D_OUT), jnp.float32),
        pltpu.VMEM_SHARED((NPAD, D_OUT), jnp.float32),
        pltpu.VMEM_SHARED((N, D_OUT), jnp.float32),
        [pltpu.SemaphoreType.DMA] * NBUF,
    ]
    out_type = jax.ShapeDtypeStruct((NC, NPAD, D_OUT), jnp.float32)

    @functools.partial(
        pl.kernel, out_type=out_type,
        mesh=plsc.VectorSubcoreMesh(**_MESH), scratch_types=scratch,
        compiler_params=pltpu.CompilerParams(use_tc_tiling_on_sc=False))
    def k(src_h, dst_h, y_h, z2_h, agg_h, srcv, dstv, rows, acc, y_s, sems):
        c = lax.axis_index("c")
        s = lax.axis_index("s")
        w = c * NS + s
        lo = s * ROWS_PT
        yl = s * (N // NS)
        pltpu.sync_copy(z2_h.at[pl.ds(lo, ROWS_PT)], acc.at[pl.ds(lo, ROWS_PT)])
        pltpu.sync_copy(y_h.at[pl.ds(yl, N // NS)], y_s.at[pl.ds(yl, N // NS)])
        pltpu.sync_copy(src_h.at[pl.ds(w * KPW, KPW)], srcv)
        pltpu.sync_copy(dst_h.at[pl.ds(w * KPW, KPW)], dstv)
        plsc.subcore_barrier()

        for b in range(NBUF):
            pltpu.async_copy(y_s.at[srcv.at[b]], rows.at[b], sems[b])

        def group(g, carry):
            for b in range(NBUF):
                j = g * NBUF + b
                pltpu.make_async_copy(y_s.at[pl.ds(0, MC)], rows.at[b],
                                      sems[b]).wait()
                pltpu.sync_copy(rows.at[b], acc.at[dstv.at[j]], add=True)

                @pl.when(j + NBUF < KPW)
                def _():
                    pltpu.async_copy(y_s.at[srcv.at[j + NBUF]], rows.at[b],
                                     sems[b])
            return carry

        lax.fori_loop(0, KPW // NBUF, group, 0)
        plsc.subcore_barrier()
        pltpu.sync_copy(acc.at[pl.ds(lo, ROWS_PT)],
                        agg_h.at[c, pl.ds(lo, ROWS_PT)])

    return k


_DN = (((1,), (1,)), ((), ()))


def _mm1_body(x_ref, wl_ref, wr_ref, y_ref, r_ref):
    xb = x_ref[...]
    y_ref[...] = lax.dot_general(xb, wl_ref[...], _DN,
                                 preferred_element_type=jnp.float32)
    r_ref[...] = lax.dot_general(xb, wr_ref[...], _DN,
                                 preferred_element_type=jnp.float32)


def _comb1_body(a0, a1, d0, d1, xr, wl2, wr2, bl1, y2_ref, hr_ref):
    deg = d0[...] + d1[...]
    iv = 1.0 / jnp.maximum(deg, 1.0)
    h = (a0[...][0] + a1[...][0]) * iv + bl1[...] + xr[...]
    h = jnp.maximum(h, 0.0)
    y2_ref[...] = lax.dot_general(h, wl2[...], _DN,
                                  preferred_element_type=jnp.float32)
    hr_ref[...] = lax.dot_general(h, wr2[...], _DN,
                                  preferred_element_type=jnp.float32)


def _comb2_body(a0, a1, d0, d1, hr, bl2, out_ref):
    deg = d0[...] + d1[...]
    iv = 1.0 / jnp.maximum(deg, 1.0)
    out_ref[...] = (a0[...][0] + a1[...][0]) * iv + bl2[...] + hr[...]


_RB = 2000


def _row_spec(d):
    return pl.BlockSpec((_RB, d), lambda i: (i, 0))


def _plane_spec(p, d):
    return pl.BlockSpec((1, _RB, d), lambda i, _p=p: (_p, i, 0))


def _full_spec(shape):
    nd = len(shape)
    return pl.BlockSpec(shape, lambda i, _n=nd: (0,) * _n)


def kernel(x, edge_index, W_l1, b_l1, W_r1, W_l2, b_l2, W_r2):
    pad = EPAD - E
    srcp = jnp.concatenate(
        [edge_index[0], jnp.zeros((pad,), jnp.int32)]).reshape(NCHUNKS_PAD, MC)
    dstp = jnp.concatenate(
        [edge_index[1], jnp.full((pad,), N, jnp.int32)]).reshape(NCHUNKS_PAD, MC)
    z2 = jnp.zeros((NPAD, D_H), jnp.float32)
    z1 = jnp.zeros((NPAD,), jnp.float32)
    z2b = jnp.zeros((NPAD, D_OUT), jnp.float32)

    y1, xr1 = pl.pallas_call(
        _mm1_body,
        grid=(N // _RB,),
        in_specs=[_row_spec(D_IN), _full_spec((D_H, D_IN)),
                  _full_spec((D_H, D_IN))],
        out_specs=[_row_spec(D_H), _row_spec(D_H)],
        out_shape=[jax.ShapeDtypeStruct((N, D_H), jnp.float32)] * 2,
    )(x, W_l1, W_r1)

    agg1, deg_flat = _sc_agg_l1()(srcp, dstp, y1, z2, z1)
    deg = deg_flat.reshape(NC, NPAD)
    d0 = deg[0, :N, None]
    d1 = deg[1, :N, None]

    y2, hr2 = pl.pallas_call(
        _comb1_body,
        grid=(N // _RB,),
        in_specs=[_plane_spec(0, D_H), _plane_spec(1, D_H),
                  pl.BlockSpec((_RB, 1), lambda i: (i, 0)),
                  pl.BlockSpec((_RB, 1), lambda i: (i, 0)),
                  _row_spec(D_H), _full_spec((D_OUT, D_H)),
                  _full_spec((D_OUT, D_H)), _full_spec((1, D_H))],
        out_specs=[_row_spec(D_OUT), _row_spec(D_OUT)],
        out_shape=[jax.ShapeDtypeStruct((N, D_OUT), jnp.float32)] * 2,
    )(agg1, agg1, d0, d1, xr1, W_l2, W_r2, b_l1.reshape(1, D_H))

    agg2 = _sc_agg_l2()(srcp, dstp, y2, z2b)

    out = pl.pallas_call(
        _comb2_body,
        grid=(N // _RB,),
        in_specs=[_plane_spec(0, D_OUT), _plane_spec(1, D_OUT),
                  pl.BlockSpec((_RB, 1), lambda i: (i, 0)),
                  pl.BlockSpec((_RB, 1), lambda i: (i, 0)),
                  _row_spec(D_OUT), _full_spec((1, D_OUT))],
        out_specs=_row_spec(D_OUT),
        out_shape=jax.ShapeDtypeStruct((N, D_OUT), jnp.float32),
    )(agg2, agg2, d0, d1, hr2, b_l2.reshape(1, D_OUT))

    return out


# speedup vs baseline: 8.9951x; 1.0039x over previous
import functools

import jax
import jax.numpy as jnp
from jax import lax
from jax.experimental import pallas as pl
from jax.experimental.pallas import tpu as pltpu
from jax.experimental.pallas import tpu_sc as plsc

N = 10000
E = 320000
D_IN = 128
D_H = 64
D_OUT = 32

NC, NS = 2, 16
NW = NC * NS
MC = 128
KPW = 80
NCHUNKS = NW * KPW
KPW0 = 132
KPW1 = 28
NCHUNKS_PAD = NS * KPW0 + (NS - 1) * KPW1 + KPW0
EPAD = NCHUNKS_PAD * MC
ROWS_PT = 632
NPAD = NS * ROWS_PT
NBUF = 4

_MESH = dict(core_axis_name="c", subcore_axis_name="s", num_cores=NC,
             num_subcores=NS)


def _sc_agg_l1():
    scratch = [
        pltpu.VMEM((KPW0, MC), jnp.int32),
        pltpu.VMEM((KPW0, MC), jnp.int32),
        pltpu.VMEM((NBUF, MC, D_H), jnp.float32),
        pltpu.VMEM((MC,), jnp.float32),
        pltpu.VMEM((ROWS_PT,), jnp.float32),
        pltpu.VMEM_SHARED((NPAD, D_H), jnp.float32),
        pltpu.VMEM_SHARED((NPAD,), jnp.float32),
        [pltpu.SemaphoreType.DMA] * NBUF,
    ]
    out_type = (jax.ShapeDtypeStruct((NC, NPAD, D_H), jnp.float32),
                jax.ShapeDtypeStruct((NW, ROWS_PT), jnp.float32))

    @functools.partial(
        pl.kernel, out_type=out_type,
        mesh=plsc.VectorSubcoreMesh(**_MESH), scratch_types=scratch,
        compiler_params=pltpu.CompilerParams(use_tc_tiling_on_sc=False))
    def k(src_h, dst_h, y_h, z2_h, agg_h, deg_h,
          srcv, dstv, rows, ones, dtmp, acc, dacc, sems):
        c = lax.axis_index("c")
        s = lax.axis_index("s")
        w = c * NS + s
        lo = s * ROWS_PT
        base = jnp.where(c == 0, s * KPW0, NS * KPW0 + s * KPW1)
        nck = jnp.where(c == 0, KPW0, KPW1)
        pltpu.sync_copy(z2_h.at[pl.ds(lo, ROWS_PT)], acc.at[pl.ds(lo, ROWS_PT)])
        for i in range(ROWS_PT // 16):
            dtmp[pl.ds(i * 16, 16)] = jnp.zeros((16,), jnp.float32)
        dtmp[pl.ds(ROWS_PT - 16, 16)] = jnp.zeros((16,), jnp.float32)
        pltpu.sync_copy(dtmp, dacc.at[pl.ds(lo, ROWS_PT)])
        pltpu.sync_copy(src_h.at[pl.ds(base, KPW0)], srcv)
        pltpu.sync_copy(dst_h.at[pl.ds(base, KPW0)], dstv)
        for i in range(MC // 16):
            ones[pl.ds(i * 16, 16)] = jnp.ones((16,), jnp.float32)
        plsc.subcore_barrier()

        for b in range(NBUF):
            pltpu.async_copy(y_h.at[srcv.at[b]], rows.at[b], sems[b])

        def group(g, carry):
            for b in range(NBUF):
                j = g * NBUF + b
                pltpu.make_async_copy(y_h.at[pl.ds(0, MC)], rows.at[b],
                                      sems[b]).wait()
                pltpu.sync_copy(rows.at[b], acc.at[dstv.at[j]], add=True)
                pltpu.sync_copy(ones, dacc.at[dstv.at[j]], add=True)

                @pl.when(j + NBUF < nck)
                def _():
                    pltpu.async_copy(y_h.at[srcv.at[j + NBUF]], rows.at[b],
                                     sems[b])
            return carry

        lax.fori_loop(0, nck // NBUF, group, 0)
        plsc.subcore_barrier()
        pltpu.sync_copy(acc.at[pl.ds(lo, ROWS_PT)],
                        agg_h.at[c, pl.ds(lo, ROWS_PT)])
        pltpu.sync_copy(dacc.at[pl.ds(lo, ROWS_PT)], dtmp)
        pltpu.sync_copy(dtmp, deg_h.at[w])

    return k


def _sc_agg_l2():
    scratch = [
        pltpu.VMEM((KPW, MC), jnp.int32),
        pltpu.VMEM((KPW, MC), jnp.int32),
        pltpu.VMEM((NBUF, MC, D_OUT), jnp.float32),
        pltpu.VMEM_SHARED((NPAD, D_OUT), jnp.float32),
        pltpu.VMEM_SHARED((N, D_OUT), jnp.float32),
        [pltpu.SemaphoreType.DMA] * NBUF,
    ]
    out_type = jax.ShapeDtypeStruct((NC, NPAD, D_OUT), jnp.float32)

    @functools.partial(
        pl.kernel, out_type=out_type,
        mesh=plsc.VectorSubcoreMesh(**_MESH), scratch_types=scratch,
        compiler_params=pltpu.CompilerParams(use_tc_tiling_on_sc=False))
    def k(src_h, dst_h, y_h, z2_h, agg_h, srcv, dstv, rows, acc, y_s, sems):
        c = lax.axis_index("c")
        s = lax.axis_index("s")
        w = c * NS + s
        lo = s * ROWS_PT
        yl = s * (N // NS)
        pltpu.sync_copy(z2_h.at[pl.ds(lo, ROWS_PT)], acc.at[pl.ds(lo, ROWS_PT)])
        pltpu.sync_copy(y_h.at[pl.ds(yl, N // NS)], y_s.at[pl.ds(yl, N // NS)])
        pltpu.sync_copy(src_h.at[pl.ds(w * KPW, KPW)], srcv)
        pltpu.sync_copy(dst_h.at[pl.ds(w * KPW, KPW)], dstv)
        plsc.subcore_barrier()

        for b in range(NBUF):
            pltpu.async_copy(y_s.at[srcv.at[b]], rows.at[b], sems[b])

        def group(g, carry):
            for b in range(NBUF):
                j = g * NBUF + b
                pltpu.make_async_copy(y_s.at[pl.ds(0, MC)], rows.at[b],
                                      sems[b]).wait()
                pltpu.sync_copy(rows.at[b], acc.at[dstv.at[j]], add=True)

                @pl.when(j + NBUF < KPW)
                def _():
                    pltpu.async_copy(y_s.at[srcv.at[j + NBUF]], rows.at[b],
                                     sems[b])
            return carry

        lax.fori_loop(0, KPW // NBUF, group, 0)
        plsc.subcore_barrier()
        pltpu.sync_copy(acc.at[pl.ds(lo, ROWS_PT)],
                        agg_h.at[c, pl.ds(lo, ROWS_PT)])

    return k


_DN = (((1,), (1,)), ((), ()))


def _mm1_body(x_ref, wl_ref, wr_ref, y_ref, r_ref):
    xb = x_ref[...]
    y_ref[...] = lax.dot_general(xb, wl_ref[...], _DN,
                                 preferred_element_type=jnp.float32)
    r_ref[...] = lax.dot_general(xb, wr_ref[...], _DN,
                                 preferred_element_type=jnp.float32)


def _comb1_body(a0, a1, d0, d1, xr, wl2, wr2, bl1, y2_ref, hr_ref):
    deg = d0[...] + d1[...]
    iv = 1.0 / jnp.maximum(deg, 1.0)
    h = (a0[...][0] + a1[...][0]) * iv + bl1[...] + xr[...]
    h = jnp.maximum(h, 0.0)
    y2_ref[...] = lax.dot_general(h, wl2[...], _DN,
                                  preferred_element_type=jnp.float32)
    hr_ref[...] = lax.dot_general(h, wr2[...], _DN,
                                  preferred_element_type=jnp.float32)


def _comb2_body(a0, a1, d0, d1, hr, bl2, out_ref):
    deg = d0[...] + d1[...]
    iv = 1.0 / jnp.maximum(deg, 1.0)
    out_ref[...] = (a0[...][0] + a1[...][0]) * iv + bl2[...] + hr[...]


_RB = 2000


def _row_spec(d):
    return pl.BlockSpec((_RB, d), lambda i: (i, 0))


def _plane_spec(p, d):
    return pl.BlockSpec((1, _RB, d), lambda i, _p=p: (_p, i, 0))


def _full_spec(shape):
    nd = len(shape)
    return pl.BlockSpec(shape, lambda i, _n=nd: (0,) * _n)


def kernel(x, edge_index, W_l1, b_l1, W_r1, W_l2, b_l2, W_r2):
    pad = EPAD - E
    srcp = jnp.concatenate(
        [edge_index[0], jnp.zeros((pad,), jnp.int32)]).reshape(NCHUNKS_PAD, MC)
    dstp = jnp.concatenate(
        [edge_index[1], jnp.full((pad,), N, jnp.int32)]).reshape(NCHUNKS_PAD, MC)
    z2 = jnp.zeros((NPAD, D_H), jnp.float32)
    z2b = jnp.zeros((NPAD, D_OUT), jnp.float32)

    y1, xr1 = pl.pallas_call(
        _mm1_body,
        grid=(N // _RB,),
        in_specs=[_row_spec(D_IN), _full_spec((D_H, D_IN)),
                  _full_spec((D_H, D_IN))],
        out_specs=[_row_spec(D_H), _row_spec(D_H)],
        out_shape=[jax.ShapeDtypeStruct((N, D_H), jnp.float32)] * 2,
    )(x, W_l1, W_r1)

    agg1, deg_rows = _sc_agg_l1()(srcp, dstp, y1, z2)
    deg = deg_rows.reshape(NC, NPAD)
    d0 = deg[0, :N, None]
    d1 = deg[1, :N, None]

    y2, hr2 = pl.pallas_call(
        _comb1_body,
        grid=(N // _RB,),
        in_specs=[_plane_spec(0, D_H), _plane_spec(1, D_H),
                  pl.BlockSpec((_RB, 1), lambda i: (i, 0)),
                  pl.BlockSpec((_RB, 1), lambda i: (i, 0)),
                  _row_spec(D_H), _full_spec((D_OUT, D_H)),
                  _full_spec((D_OUT, D_H)), _full_spec((1, D_H))],
        out_specs=[_row_spec(D_OUT), _row_spec(D_OUT)],
        out_shape=[jax.ShapeDtypeStruct((N, D_OUT), jnp.float32)] * 2,
    )(agg1, agg1, d0, d1, xr1, W_l2, W_r2, b_l1.reshape(1, D_H))

    agg2 = _sc_agg_l2()(srcp, dstp, y2, z2b)

    out = pl.pallas_call(
        _comb2_body,
        grid=(N // _RB,),
        in_specs=[_plane_spec(0, D_OUT), _plane_spec(1, D_OUT),
                  pl.BlockSpec((_RB, 1), lambda i: (i, 0)),
                  pl.BlockSpec((_RB, 1), lambda i: (i, 0)),
                  _row_spec(D_OUT), _full_spec((1, D_OUT))],
        out_specs=_row_spec(D_OUT),
        out_shape=jax.ShapeDtypeStruct((N, D_OUT), jnp.float32),
    )(agg2, agg2, d0, d1, hr2, b_l2.reshape(1, D_OUT))

    return out
